# Initial kernel scaffold; baseline (speedup 1.0000x reference)
import jax
import jax.numpy as jnp
from jax.experimental import pallas as pl


def kernel(z_e):
    raise NotImplementedError("write your pallas kernel here")


# speedup vs baseline: 11.2570x; 11.2570x over previous
import jax
import jax.numpy as jnp
from jax.experimental import pallas as pl


_B, _D, _T = 32, 256, 1024
_TC = 256


def _fsq_body(z_ref, zq_ref, idx_ref):
    z = z_ref[0]
    y = jnp.tanh(z)
    tr = jnp.minimum((8.0 - 8.0 * y).astype(jnp.int32), 15)
    idx = 15 - tr
    zq_ref[0] = 0.9375 - 0.125 * tr.astype(jnp.float32)
    idx_ref[0] = jnp.transpose(idx, (1, 0))


def kernel(z_e):
    B, D, T = z_e.shape
    grid = (B, T // _TC)
    zq, idx = pl.pallas_call(
        _fsq_body,
        grid=grid,
        in_specs=[pl.BlockSpec((1, D, _TC), lambda b, t: (b, 0, t))],
        out_specs=[
            pl.BlockSpec((1, D, _TC), lambda b, t: (b, 0, t)),
            pl.BlockSpec((1, _TC, D), lambda b, t: (b, t, 0)),
        ],
        out_shape=[
            jax.ShapeDtypeStruct((B, D, T), jnp.float32),
            jax.ShapeDtypeStruct((B, T, D), jnp.int32),
        ],
    )(z_e)
    aux_loss = jnp.asarray(0.0, dtype=z_e.dtype)
    return (zq, idx, aux_loss)
